# Initial kernel scaffold; baseline (speedup 1.0000x reference)
import jax
import jax.numpy as jnp
from jax.experimental import pallas as pl


def kernel(state):
    raise NotImplementedError("write your pallas kernel here")


# speedup vs baseline: 1.0354x; 1.0354x over previous
import jax
import jax.numpy as jnp
from jax.experimental import pallas as pl

MAX_H = 128
ROWS = 8
CHUNK = 256


def _update_kernel(state_ref, out_ref):
    h = pl.program_id(1)
    out_ref[...] = jnp.zeros_like(out_ref)

    @pl.when(h == 0)
    def _():
        out_ref[0, :, :] = jnp.mean(state_ref[...], axis=0)


def kernel(state):
    if state.ndim == 2:
        state = state[None, :, :]
    B, P, D = state.shape
    buf = pl.pallas_call(
        _update_kernel,
        grid=(P // CHUNK, MAX_H // ROWS),
        in_specs=[pl.BlockSpec((B, CHUNK, D), lambda p, h: (0, p, 0))],
        out_specs=pl.BlockSpec((ROWS, CHUNK, D), lambda p, h: (h, p, 0)),
        out_shape=jax.ShapeDtypeStruct((MAX_H, P, D), state.dtype),
    )(state)
    return buf, jnp.asarray(1, dtype=jnp.int32)


# speedup vs baseline: 1.0725x; 1.0358x over previous
import jax
import jax.numpy as jnp
from jax.experimental import pallas as pl

MAX_H = 128
ROWS = 16
CHUNK = 256


def _update_kernel(state_ref, out_ref):
    h = pl.program_id(1)
    out_ref[...] = jnp.zeros_like(out_ref)

    @pl.when(h == 0)
    def _():
        out_ref[0, :, :] = jnp.mean(state_ref[...], axis=0)


def kernel(state):
    if state.ndim == 2:
        state = state[None, :, :]
    B, P, D = state.shape
    buf = pl.pallas_call(
        _update_kernel,
        grid=(P // CHUNK, MAX_H // ROWS),
        in_specs=[pl.BlockSpec((B, CHUNK, D), lambda p, h: (0, p, 0))],
        out_specs=pl.BlockSpec((ROWS, CHUNK, D), lambda p, h: (h, p, 0)),
        out_shape=jax.ShapeDtypeStruct((MAX_H, P, D), state.dtype),
    )(state)
    return buf, jnp.asarray(1, dtype=jnp.int32)


# speedup vs baseline: 1.1328x; 1.0563x over previous
import jax
import jax.numpy as jnp
from jax.experimental import pallas as pl

MAX_H = 128
ROWS = 32
CHUNK = 256


def _update_kernel(state_ref, out_ref):
    h = pl.program_id(1)
    out_ref[...] = jnp.zeros_like(out_ref)

    @pl.when(h == 0)
    def _():
        out_ref[0, :, :] = jnp.mean(state_ref[...], axis=0)


def kernel(state):
    if state.ndim == 2:
        state = state[None, :, :]
    B, P, D = state.shape
    buf = pl.pallas_call(
        _update_kernel,
        grid=(P // CHUNK, MAX_H // ROWS),
        in_specs=[pl.BlockSpec((B, CHUNK, D), lambda p, h: (0, p, 0))],
        out_specs=pl.BlockSpec((ROWS, CHUNK, D), lambda p, h: (h, p, 0)),
        out_shape=jax.ShapeDtypeStruct((MAX_H, P, D), state.dtype),
    )(state)
    return buf, jnp.asarray(1, dtype=jnp.int32)


# speedup vs baseline: 1.1355x; 1.0024x over previous
import jax
import jax.numpy as jnp
from jax.experimental import pallas as pl

MAX_H = 128
ROWS = 64
CHUNK = 128


def _update_kernel(state_ref, out_ref):
    h = pl.program_id(1)
    out_ref[...] = jnp.zeros_like(out_ref)

    @pl.when(h == 0)
    def _():
        out_ref[0, :, :] = jnp.mean(state_ref[...], axis=0)


def kernel(state):
    if state.ndim == 2:
        state = state[None, :, :]
    B, P, D = state.shape
    buf = pl.pallas_call(
        _update_kernel,
        grid=(P // CHUNK, MAX_H // ROWS),
        in_specs=[pl.BlockSpec((B, CHUNK, D), lambda p, h: (0, p, 0))],
        out_specs=pl.BlockSpec((ROWS, CHUNK, D), lambda p, h: (h, p, 0)),
        out_shape=jax.ShapeDtypeStruct((MAX_H, P, D), state.dtype),
    )(state)
    return buf, jnp.asarray(1, dtype=jnp.int32)


# speedup vs baseline: 1.1366x; 1.0010x over previous
import jax
import jax.numpy as jnp
from jax.experimental import pallas as pl
from jax.experimental.pallas import tpu as pltpu

MAX_H = 128
ZROWS = 4


def _update_kernel(state_hbm, out_hbm, zeros_vmem, state_vmem, mean_vmem,
                   zsem, ssem, msem):
    zeros_vmem[...] = jnp.zeros_like(zeros_vmem)
    in_copy = pltpu.make_async_copy(state_hbm, state_vmem, ssem)
    in_copy.start()
    copies = []
    for s in range(1, MAX_H, ZROWS):
        r = min(ZROWS, MAX_H - s)
        c = pltpu.make_async_copy(
            zeros_vmem.at[pl.ds(0, r)], out_hbm.at[pl.ds(s, r)], zsem)
        c.start()
        copies.append(c)
    in_copy.wait()
    mean_vmem[...] = jnp.mean(state_vmem[...], axis=0, keepdims=True)
    m_copy = pltpu.make_async_copy(mean_vmem, out_hbm.at[pl.ds(0, 1)], msem)
    m_copy.start()
    for c in copies:
        c.wait()
    m_copy.wait()


def kernel(state):
    if state.ndim == 2:
        state = state[None, :, :]
    B, P, D = state.shape
    buf = pl.pallas_call(
        _update_kernel,
        in_specs=[pl.BlockSpec(memory_space=pltpu.MemorySpace.HBM)],
        out_specs=pl.BlockSpec(memory_space=pltpu.MemorySpace.HBM),
        out_shape=jax.ShapeDtypeStruct((MAX_H, P, D), state.dtype),
        scratch_shapes=[
            pltpu.VMEM((ZROWS, P, D), state.dtype),
            pltpu.VMEM((B, P, D), state.dtype),
            pltpu.VMEM((1, P, D), state.dtype),
            pltpu.SemaphoreType.DMA,
            pltpu.SemaphoreType.DMA,
            pltpu.SemaphoreType.DMA,
        ],
    )(state)
    return buf, jnp.asarray(1, dtype=jnp.int32)
